# baseline (speedup 1.0000x reference)
import functools

import jax
import jax.numpy as jnp
from jax import lax
from jax.experimental import pallas as pl
from jax.experimental.pallas import tpu as pltpu
from jax.experimental.pallas import tpu_sc as plsc

_B = 16384
_F = 26
_D = 16
_V = 1000000
_CB = 128
_NBLK = (_V + 127) // 128
_LAST = _NBLK - 1


def _make_transpose():
    info = plsc.get_sparse_core_info()
    nc, ns = info.num_cores, info.num_subcores
    nw = nc * ns
    blocks_per_w = (_NBLK + nw - 1) // nw

    mesh = plsc.VectorSubcoreMesh(core_axis_name="c", subcore_axis_name="s")

    @functools.partial(
        pl.kernel,
        mesh=mesh,
        out_type=jax.ShapeDtypeStruct((_V * _D // 128, 128), jnp.float32),
        compiler_params=pltpu.CompilerParams(
            use_tc_tiling_on_sc=True, needs_layout_passes=False
        ),
        scratch_types=[
            pltpu.VMEM((_D, 128), jnp.float32),
            pltpu.VMEM((16, 128), jnp.float32),
        ],
    )
    def transpose(tt_hbm, tail_hbm, out_hbm, in_v, out_v):
        wid = lax.axis_index("s") * nc + lax.axis_index("c")
        lanes = lax.iota(jnp.int32, 16)

        def emit_rows(nrows, r0):
            def row_body(r, carry):
                for s in range(8):
                    col = jnp.full((16,), r * 8 + s, jnp.int32)
                    seg = plsc.load_gather(in_v, [lanes, col])
                    out_v[r, pl.ds(s * _D, _D)] = seg
                return carry

            lax.fori_loop(0, nrows, row_body, 0)
            pltpu.sync_copy(out_v.at[pl.ds(0, nrows)],
                            out_hbm.at[pl.ds(r0, nrows)])

        def blk_body(k, carry):
            b = wid * blocks_per_w + k

            @pl.when(b < _LAST)
            def _full():
                pltpu.sync_copy(tt_hbm.at[pl.ds(0, _D), pl.ds(b * 128, 128)], in_v)
                emit_rows(16, b * 16)

            @pl.when(b == _LAST)
            def _tail():
                pltpu.sync_copy(tail_hbm, in_v)
                emit_rows(8, _LAST * 16)

            return carry

        lax.fori_loop(0, blocks_per_w, blk_body, 0)

    return transpose


def _make_lookup():
    info = plsc.get_sparse_core_info()
    nc, ns = info.num_cores, info.num_subcores
    nw = nc * ns
    b_per_w = _B // nw
    n_chunks = b_per_w // _CB

    mesh = plsc.VectorSubcoreMesh(core_axis_name="c", subcore_axis_name="s")

    @functools.partial(
        pl.kernel,
        mesh=mesh,
        out_type=jax.ShapeDtypeStruct((_B, _D), jnp.float32),
        compiler_params=pltpu.CompilerParams(use_tc_tiling_on_sc=False),
        scratch_types=[
            pltpu.VMEM((32, b_per_w), jnp.int32),
            pltpu.VMEM((_F * _CB, _D), jnp.float32),
            pltpu.VMEM((_CB, _D), jnp.float32),
            pltpu.SemaphoreType.DMA,
        ],
    )
    def emb_sum(idx_hbm, table_hbm, out_hbm, idx_v, rows_v, out_v, sem):
        wid = lax.axis_index("s") * nc + lax.axis_index("c")
        pltpu.sync_copy(idx_hbm.at[pl.ds(0, 32), pl.ds(wid * b_per_w, b_per_w)], idx_v)

        def chunk_body(c, carry):
            for f in range(_F):
                pltpu.async_copy(
                    table_hbm.at[idx_v.at[f, pl.ds(c * _CB, _CB)]],
                    rows_v.at[pl.ds(f * _CB, _CB)],
                    sem,
                )
            pltpu.make_async_copy(
                table_hbm.at[pl.ds(0, _F * _CB)], rows_v, sem
            ).wait()

            def reduce_body(i, inner):
                acc = rows_v[i]
                for f in range(1, _F):
                    acc = acc + rows_v[f * _CB + i]
                out_v[i] = acc
                return inner

            lax.fori_loop(0, _CB, reduce_body, 0)
            pltpu.sync_copy(out_v, out_hbm.at[pl.ds(wid * b_per_w + c * _CB, _CB)])
            return carry

        lax.fori_loop(0, n_chunks, chunk_body, 0)

    return emb_sum


def kernel(inputs, table):
    idx_t = jnp.pad(inputs.astype(jnp.int32).T, ((0, 32 - _F), (0, 0)), mode="edge")
    tt = table.T
    tail = jnp.pad(tt[:, _LAST * 128:], ((0, 0), (0, 128 - (_V - _LAST * 128))))
    table_rm = _make_transpose()(tt, tail)
    table16 = table_rm.reshape(_V, _D)
    return _make_lookup()(idx_t, table16)


# speedup vs baseline: 1.4431x; 1.4431x over previous
import functools

import jax
import jax.numpy as jnp
from jax import lax
from jax.experimental import pallas as pl
from jax.experimental.pallas import tpu as pltpu
from jax.experimental.pallas import tpu_sc as plsc

_B = 16384
_F = 26
_D = 16
_V = 1000000
_CB = 128
_NBLK = (_V + 127) // 128
_LAST = _NBLK - 1


_BC = 256
_NFULL = _V // _BC
_NBUF = 4


def _make_transpose():
    info = plsc.get_sparse_core_info()
    nc, ns = info.num_cores, info.num_subcores
    nw = nc * ns
    rem = _NFULL % nw

    mesh = plsc.VectorSubcoreMesh(core_axis_name="c", subcore_axis_name="s")

    in_bufs = [pltpu.VMEM((_D, _BC), jnp.float32) for _ in range(_NBUF)]
    out_bufs = [pltpu.VMEM((_BC // 8, 128), jnp.float32) for _ in range(_NBUF)]

    @functools.partial(
        pl.kernel,
        mesh=mesh,
        out_type=jax.ShapeDtypeStruct((_V * _D // 128, 128), jnp.float32),
        compiler_params=pltpu.CompilerParams(
            use_tc_tiling_on_sc=True, needs_layout_passes=False
        ),
        scratch_types=in_bufs + out_bufs + [
            pltpu.SemaphoreType.DMA,
            pltpu.SemaphoreType.DMA,
        ],
    )
    def transpose(tt_hbm, tail_hbm, out_hbm,
                  i0, i1, i2, i3, o0, o1, o2, o3, sem_in, sem_out):
        wid = lax.axis_index("s") * nc + lax.axis_index("c")
        lanes = lax.iota(jnp.int32, 16)
        ins = [i0, i1, i2, i3]
        outs = [o0, o1, o2, o3]
        n_w = jnp.where(wid < rem, _NFULL // nw + 1, _NFULL // nw)

        def fire_in(k, buf):
            @pl.when(k < n_w)
            def _():
                c0 = (k * nw + wid) * _BC
                pltpu.async_copy(tt_hbm.at[pl.ds(0, _D), pl.ds(c0, _BC)], buf,
                                 sem_in)

        def compute(k, src, dst):
            def row_body(r, carry):
                for s in range(8):
                    col = jnp.full((16,), r * 8 + s, jnp.int32)
                    dst[r, pl.ds(s * _D, _D)] = plsc.load_gather(src, [lanes, col])
                return carry

            lax.fori_loop(0, _BC // 8, row_body, 0)

        for b in range(_NBUF):
            fire_in(b, ins[b])

        def outer(kk, carry):
            for b in range(_NBUF):
                k = kk * _NBUF + b

                @pl.when(k < n_w)
                def _(k=k, b=b):
                    pltpu.make_async_copy(
                        tt_hbm.at[pl.ds(0, _D), pl.ds(0, _BC)], ins[b], sem_in
                    ).wait()

                    @pl.when(k >= _NBUF)
                    def _():
                        pltpu.make_async_copy(
                            outs[b], out_hbm.at[pl.ds(0, _BC // 8)], sem_out
                        ).wait()

                    compute(k, ins[b], outs[b])
                    r0 = (k * nw + wid) * (_BC // 8)
                    pltpu.async_copy(outs[b], out_hbm.at[pl.ds(r0, _BC // 8)],
                                     sem_out)
                    fire_in(k + _NBUF, ins[b])

            return carry

        n_outer = (_NFULL // nw + 1 + _NBUF - 1) // _NBUF
        lax.fori_loop(0, n_outer, outer, 0)

        for b in range(_NBUF):
            pltpu.make_async_copy(
                outs[b], out_hbm.at[pl.ds(0, _BC // 8)], sem_out
            ).wait()

        @pl.when(wid == 0)
        def _tail():
            pltpu.sync_copy(tail_hbm, i0.at[pl.ds(0, _D), pl.ds(0, 128)])

            def row_body(r, carry):
                for s in range(8):
                    col = jnp.full((16,), r * 8 + s, jnp.int32)
                    o0[r, pl.ds(s * _D, _D)] = plsc.load_gather(i0, [lanes, col])
                return carry

            lax.fori_loop(0, 8, row_body, 0)
            pltpu.sync_copy(o0.at[pl.ds(0, 8)],
                            out_hbm.at[pl.ds((_V // 8) - 8, 8)])

    return transpose


def _make_lookup():
    info = plsc.get_sparse_core_info()
    nc, ns = info.num_cores, info.num_subcores
    nw = nc * ns
    b_per_w = _B // nw
    n_chunks = b_per_w // _CB

    mesh = plsc.VectorSubcoreMesh(core_axis_name="c", subcore_axis_name="s")

    @functools.partial(
        pl.kernel,
        mesh=mesh,
        out_type=jax.ShapeDtypeStruct((_B, _D), jnp.float32),
        compiler_params=pltpu.CompilerParams(use_tc_tiling_on_sc=False),
        scratch_types=[
            pltpu.VMEM((32, b_per_w), jnp.int32),
            pltpu.VMEM((_F * _CB, _D), jnp.float32),
            pltpu.VMEM((_CB, _D), jnp.float32),
            pltpu.SemaphoreType.DMA,
        ],
    )
    def emb_sum(idx_hbm, table_hbm, out_hbm, idx_v, rows_v, out_v, sem):
        wid = lax.axis_index("s") * nc + lax.axis_index("c")
        pltpu.sync_copy(idx_hbm.at[pl.ds(0, 32), pl.ds(wid * b_per_w, b_per_w)], idx_v)

        def chunk_body(c, carry):
            for f in range(_F):
                pltpu.async_copy(
                    table_hbm.at[idx_v.at[f, pl.ds(c * _CB, _CB)]],
                    rows_v.at[pl.ds(f * _CB, _CB)],
                    sem,
                )
            pltpu.make_async_copy(
                table_hbm.at[pl.ds(0, _F * _CB)], rows_v, sem
            ).wait()

            def reduce_body(i, inner):
                acc = rows_v[i]
                for f in range(1, _F):
                    acc = acc + rows_v[f * _CB + i]
                out_v[i] = acc
                return inner

            lax.fori_loop(0, _CB, reduce_body, 0)
            pltpu.sync_copy(out_v, out_hbm.at[pl.ds(wid * b_per_w + c * _CB, _CB)])
            return carry

        lax.fori_loop(0, n_chunks, chunk_body, 0)

    return emb_sum


def kernel(inputs, table):
    idx_t = jnp.pad(inputs.astype(jnp.int32).T, ((0, 32 - _F), (0, 0)), mode="edge")
    tt = table.T
    tail = jnp.pad(tt[:, _LAST * 128:], ((0, 0), (0, 128 - (_V - _LAST * 128))))
    table_rm = _make_transpose()(tt, tail)
    table16 = table_rm.reshape(_V, _D)
    return _make_lookup()(idx_t, table16)


# speedup vs baseline: 2.7822x; 1.9279x over previous
import functools

import jax
import jax.numpy as jnp
from jax import lax
from jax.experimental import pallas as pl
from jax.experimental.pallas import tpu as pltpu
from jax.experimental.pallas import tpu_sc as plsc

_B = 16384
_F = 26
_D = 16
_V = 1000000
_CB = 128
_NBLK = (_V + 127) // 128
_LAST = _NBLK - 1


_BC = 256
_NFULL = _V // _BC
_NBUF = 4


def _make_transpose():
    info = plsc.get_sparse_core_info()
    nc, ns = info.num_cores, info.num_subcores
    nw = nc * ns
    rem = _NFULL % nw

    mesh = plsc.VectorSubcoreMesh(core_axis_name="c", subcore_axis_name="s")

    in_bufs = [pltpu.VMEM((_D, _BC), jnp.float32) for _ in range(_NBUF)]
    out_bufs = [pltpu.VMEM((_BC // 8, 128), jnp.float32) for _ in range(_NBUF)]

    @functools.partial(
        pl.kernel,
        mesh=mesh,
        out_type=jax.ShapeDtypeStruct((_V * _D // 128, 128), jnp.float32),
        compiler_params=pltpu.CompilerParams(
            use_tc_tiling_on_sc=True, needs_layout_passes=False
        ),
        scratch_types=in_bufs + out_bufs + [
            pltpu.SemaphoreType.DMA,
            pltpu.SemaphoreType.DMA,
        ],
    )
    def transpose(tt_hbm, tail_hbm, out_hbm,
                  i0, i1, i2, i3, o0, o1, o2, o3, sem_in, sem_out):
        wid = lax.axis_index("s") * nc + lax.axis_index("c")
        lanes = lax.iota(jnp.int32, 16)
        ins = [i0, i1, i2, i3]
        outs = [o0, o1, o2, o3]
        n_w = jnp.where(wid < rem, _NFULL // nw + 1, _NFULL // nw)

        def fire_in(k, buf):
            @pl.when(k < n_w)
            def _():
                c0 = (k * nw + wid) * _BC
                pltpu.async_copy(tt_hbm.at[pl.ds(0, _D), pl.ds(c0, _BC)], buf,
                                 sem_in)

        def compute(k, src, dst):
            def grp_body(g, carry):
                c0 = g * 16
                for j in range(16):
                    kdiag = (16 - j) & 15
                    diag = plsc.load_gather(
                        src, [lanes, c0 + ((lanes + kdiag) & 15)]
                    )
                    rot = diag.at[(lanes + j) & 15].get(mode="promise_in_bounds")
                    plsc.store_scatter(
                        dst,
                        [2 * g + (lanes >> 3),
                         ((lanes & 7) << 4) + ((lanes + j) & 15)],
                        rot,
                    )
                return carry

            lax.fori_loop(0, _BC // 16, grp_body, 0)

        for b in range(_NBUF):
            fire_in(b, ins[b])

        def outer(kk, carry):
            for b in range(_NBUF):
                k = kk * _NBUF + b

                @pl.when(k < n_w)
                def _(k=k, b=b):
                    pltpu.make_async_copy(
                        tt_hbm.at[pl.ds(0, _D), pl.ds(0, _BC)], ins[b], sem_in
                    ).wait()

                    @pl.when(k >= _NBUF)
                    def _():
                        pltpu.make_async_copy(
                            outs[b], out_hbm.at[pl.ds(0, _BC // 8)], sem_out
                        ).wait()

                    compute(k, ins[b], outs[b])
                    r0 = (k * nw + wid) * (_BC // 8)
                    pltpu.async_copy(outs[b], out_hbm.at[pl.ds(r0, _BC // 8)],
                                     sem_out)
                    fire_in(k + _NBUF, ins[b])

            return carry

        n_outer = (_NFULL // nw + 1 + _NBUF - 1) // _NBUF
        lax.fori_loop(0, n_outer, outer, 0)

        for b in range(_NBUF):
            pltpu.make_async_copy(
                outs[b], out_hbm.at[pl.ds(0, _BC // 8)], sem_out
            ).wait()

        @pl.when(wid == 0)
        def _tail():
            pltpu.sync_copy(tail_hbm, i0.at[pl.ds(0, _D), pl.ds(0, 128)])

            def row_body(r, carry):
                for s in range(8):
                    col = jnp.full((16,), r * 8 + s, jnp.int32)
                    o0[r, pl.ds(s * _D, _D)] = plsc.load_gather(i0, [lanes, col])
                return carry

            lax.fori_loop(0, 8, row_body, 0)
            pltpu.sync_copy(o0.at[pl.ds(0, 8)],
                            out_hbm.at[pl.ds((_V // 8) - 8, 8)])

    return transpose


def _make_lookup():
    info = plsc.get_sparse_core_info()
    nc, ns = info.num_cores, info.num_subcores
    nw = nc * ns
    b_per_w = _B // nw
    n_chunks = b_per_w // _CB

    mesh = plsc.VectorSubcoreMesh(core_axis_name="c", subcore_axis_name="s")

    @functools.partial(
        pl.kernel,
        mesh=mesh,
        out_type=jax.ShapeDtypeStruct((_B, _D), jnp.float32),
        compiler_params=pltpu.CompilerParams(use_tc_tiling_on_sc=False),
        scratch_types=[
            pltpu.VMEM((32, b_per_w), jnp.int32),
            pltpu.VMEM((_F * _CB, _D), jnp.float32),
            pltpu.VMEM((_CB, _D), jnp.float32),
            pltpu.SemaphoreType.DMA,
        ],
    )
    def emb_sum(idx_hbm, table_hbm, out_hbm, idx_v, rows_v, out_v, sem):
        wid = lax.axis_index("s") * nc + lax.axis_index("c")
        pltpu.sync_copy(idx_hbm.at[pl.ds(0, 32), pl.ds(wid * b_per_w, b_per_w)], idx_v)

        def chunk_body(c, carry):
            for f in range(_F):
                pltpu.async_copy(
                    table_hbm.at[idx_v.at[f, pl.ds(c * _CB, _CB)]],
                    rows_v.at[pl.ds(f * _CB, _CB)],
                    sem,
                )
            pltpu.make_async_copy(
                table_hbm.at[pl.ds(0, _F * _CB)], rows_v, sem
            ).wait()

            def reduce_body(i, inner):
                acc = rows_v[i]
                for f in range(1, _F):
                    acc = acc + rows_v[f * _CB + i]
                out_v[i] = acc
                return inner

            lax.fori_loop(0, _CB, reduce_body, 0)
            pltpu.sync_copy(out_v, out_hbm.at[pl.ds(wid * b_per_w + c * _CB, _CB)])
            return carry

        lax.fori_loop(0, n_chunks, chunk_body, 0)

    return emb_sum


def kernel(inputs, table):
    idx_t = jnp.pad(inputs.astype(jnp.int32).T, ((0, 32 - _F), (0, 0)), mode="edge")
    tt = table.T
    tail = jnp.pad(tt[:, _LAST * 128:], ((0, 0), (0, 128 - (_V - _LAST * 128))))
    table_rm = _make_transpose()(tt, tail)
    table16 = table_rm.reshape(_V, _D)
    return _make_lookup()(idx_t, table16)
